# baseline (speedup 1.0000x reference)
import functools

import jax
import jax.numpy as jnp
from jax import lax
from jax.experimental import pallas as pl
from jax.experimental.pallas import tpu as pltpu
from jax.experimental.pallas import tpu_sc as plsc

X = Y = Z = 64
YZ = Y * Z
NB = 2
CHUNK = 6


def kernel(img, index_flat_inv):
    del index_flat_inv
    B, C, N = img.shape
    planes = B * C * X
    flat = img.reshape(planes * YZ)

    n_workers = 32
    per_w = planes // n_workers
    chunks = per_w // CHUNK
    cwords = CHUNK * YZ

    mesh = plsc.VectorSubcoreMesh(core_axis_name="c", subcore_axis_name="s")

    @functools.partial(
        pl.kernel,
        mesh=mesh,
        out_type=jax.ShapeDtypeStruct((planes * YZ,), jnp.float32),
        scratch_types=[
            pltpu.VMEM_SHARED((16, NB, cwords), jnp.float32),
            pltpu.VMEM((NB, cwords), jnp.float32),
        ] + [pltpu.SemaphoreType.DMA] * (2 * NB),
    )
    def run(img_hbm, out_hbm, spm, tsp, *sems):
        sid = lax.axis_index("s")
        sis = sems[:NB]
        sos = sems[NB:]

        for b in range(NB):
            pltpu.async_copy(spm.at[sid, b], tsp.at[b], sis[b])

        def outer(o, carry):
            for b in range(NB):
                i = NB * o + b
                pltpu.make_async_copy(spm.at[sid, b], tsp.at[b], sis[b]).wait()

                @pl.when(i >= NB)
                def _():
                    pltpu.make_async_copy(tsp.at[b], spm.at[sid, b], sos[b]).wait()

                pltpu.async_copy(tsp.at[b], spm.at[sid, b], sos[b])

                @pl.when(i + NB < chunks)
                def _():
                    pltpu.async_copy(spm.at[sid, b], tsp.at[b], sis[b])

            return carry

        lax.fori_loop(0, chunks // NB, outer, 0)

        for b in range(NB):
            pltpu.make_async_copy(tsp.at[b], spm.at[sid, b], sos[b]).wait()

        wid = sid * 2 + lax.axis_index("c")
        pltpu.sync_copy(tsp.at[0], out_hbm.at[pl.ds(wid * cwords, cwords)])

    out = run(flat)
    return out.reshape(B, C, N)
